# Initial kernel scaffold; baseline (speedup 1.0000x reference)
import jax
import jax.numpy as jnp
from jax.experimental import pallas as pl


def kernel(x, node_types, W, b):
    raise NotImplementedError("write your pallas kernel here")


# speedup vs baseline: 2.0719x; 2.0719x over previous
import jax
import jax.numpy as jnp
from jax.experimental import pallas as pl


def _body(x_ref, t_ref, w_ref, b_ref, o_ref):
    xb = x_ref[...]
    tb = t_ref[...]
    T = w_ref.shape[0]
    acc = None
    for t in range(T):
        p = jnp.dot(xb, w_ref[t], preferred_element_type=jnp.float32)
        p = p + b_ref[t][None, :]
        if acc is None:
            acc = p
        else:
            acc = jnp.where(tb == t, p, acc)
    o_ref[...] = acc


def kernel(x, node_types, W, b):
    N, D = x.shape
    T, H, _ = W.shape
    B = 1000
    assert N % B == 0
    nt2 = node_types.astype(jnp.int32).reshape(N, 1)
    Wt = jnp.swapaxes(W, 1, 2)
    return pl.pallas_call(
        _body,
        grid=(N // B,),
        in_specs=[
            pl.BlockSpec((B, D), lambda i: (i, 0)),
            pl.BlockSpec((B, 1), lambda i: (i, 0)),
            pl.BlockSpec((T, D, H), lambda i: (0, 0, 0)),
            pl.BlockSpec((T, H), lambda i: (0, 0)),
        ],
        out_specs=pl.BlockSpec((B, H), lambda i: (i, 0)),
        out_shape=jax.ShapeDtypeStruct((N, H), x.dtype),
    )(x, nt2, Wt, b)
